# baseline (speedup 1.0000x reference)
import functools

import jax
import jax.numpy as jnp
from jax import lax
from jax.experimental import pallas as pl
from jax.experimental.pallas import tpu as pltpu
from jax.experimental.pallas import tpu_sc as plsc

PADDING_ID = 0
NUM_CORES = 2
NUM_SUBCORES = 16
LANES = 16
NUM_WORKERS = NUM_CORES * NUM_SUBCORES
NBUF = 2
CHUNK = 16
UNROLL = 16


@functools.cache
def _build(batch: int, seq_len: int, d: int):
    assert seq_len % NUM_WORKERS == 0
    span = seq_len // NUM_WORKERS
    assert span % CHUNK == 0 and d % LANES == 0
    n_sub = span // CHUNK
    n_chunks = batch * n_sub
    assert n_chunks >= NBUF

    mesh = plsc.VectorSubcoreMesh(core_axis_name="c", subcore_axis_name="s")

    scratch = (
        [pltpu.VMEM((CHUNK,), jnp.int32) for _ in range(NBUF)]
        + [pltpu.VMEM((CHUNK, d), jnp.float32) for _ in range(NBUF)]
        + [pltpu.VMEM((span + 8, d), jnp.float32)]
        + [pltpu.VMEM((CHUNK * LANES,), jnp.float32)]
        + [pltpu.SemaphoreType.DMA] * (2 * NBUF + 1)
    )

    @functools.partial(
        pl.kernel,
        out_type=jax.ShapeDtypeStruct((batch * seq_len, d), jnp.float32),
        mesh=mesh,
        scratch_types=scratch,
        compiler_params=pltpu.CompilerParams(needs_layout_passes=False),
    )
    def k(enc_hbm, emb_hbm, pe_hbm, out_hbm, *bufs):
        ids_v = bufs[0:NBUF]
        tok_v = bufs[NBUF : 2 * NBUF]
        peb = bufs[2 * NBUF]
        mrep = bufs[2 * NBUF + 1]
        sems = bufs[2 * NBUF + 2 :]
        tok_sem = sems[0:NBUF]
        st_sem = sems[NBUF : 2 * NBUF]
        pe_sem = sems[2 * NBUF]

        wid = lax.axis_index("s") * NUM_CORES + lax.axis_index("c")
        l_base = wid * span

        pe_cp = pltpu.async_copy(pe_hbm.at[pl.ds(l_base, span + 8)], peb, pe_sem)

        def chunk_off(ci):
            b_row, s = ci // n_sub, ci % n_sub
            return b_row * seq_len + l_base + s * CHUNK

        tok_cp = [None] * n_chunks
        st_cp = [None] * n_chunks

        def issue_gather(ci):
            b = ci % NBUF
            pltpu.sync_copy(enc_hbm.at[pl.ds(chunk_off(ci), CHUNK)], ids_v[b])
            tok_cp[ci] = pltpu.async_copy(
                emb_hbm.at[ids_v[b]], tok_v[b], tok_sem[b]
            )

        issue_gather(0)
        pe_cp.wait()

        for ci in range(n_chunks):
            b = ci % NBUF
            s = ci % n_sub
            if ci + 1 < n_chunks:
                if st_cp[ci + 1 - NBUF] is not None:
                    st_cp[ci + 1 - NBUF].wait()
                issue_gather(ci + 1)
            tok_cp[ci].wait()

            ids = ids_v[b][pl.ds(0, LANES)]
            padf = jnp.where(ids == PADDING_ID, 1.0, 0.0).astype(jnp.float32)
            rows16 = lax.broadcasted_iota(jnp.int32, (LANES,), 0)
            for col in range(LANES):
                plsc.store_scatter(mrep, [rows16 * LANES + col], padf)
            pe0 = jnp.where(
                lax.rem(rows16, 2) == 0, 0.0, 1.0
            ).astype(jnp.float32)

            def row_add(i, carry, _b=b, _s=s):
                m = mrep[pl.ds(i * LANES, LANES)]

                def col_grp(g, cc):
                    for u in range(UNROLL):
                        sl = pl.ds(g * (UNROLL * LANES) + u * LANES, LANES)
                        pr = peb[_s * CHUNK + i + 1, sl]
                        tok_v[_b][i, sl] = tok_v[_b][i, sl] + (
                            pr + m * (pe0 - pr)
                        )
                    return cc

                return lax.fori_loop(0, d // LANES // UNROLL, col_grp, carry)

            lax.fori_loop(0, CHUNK, row_add, 0)

            st_cp[ci] = pltpu.async_copy(
                tok_v[b], out_hbm.at[pl.ds(chunk_off(ci), CHUNK)], st_sem[b]
            )

        for ci in range(n_chunks - NBUF, n_chunks):
            st_cp[ci].wait()

    return k


def kernel(enc_input, src_emb, pe_table):
    b, l = enc_input.shape
    d = src_emb.shape[1]
    flat = enc_input.reshape(b * l)
    out = _build(b, l, d)(flat, src_emb, pe_table)
    return out.reshape(b, l, d)


# speedup vs baseline: 1.5131x; 1.5131x over previous
import functools

import jax
import jax.numpy as jnp
from jax import lax
from jax.experimental import pallas as pl
from jax.experimental.pallas import tpu as pltpu
from jax.experimental.pallas import tpu_sc as plsc

PADDING_ID = 0
NUM_CORES = 2
NUM_SUBCORES = 16
LANES = 16
NUM_WORKERS = NUM_CORES * NUM_SUBCORES
NBUF = 2
CHUNK = 16
UNROLL = 16


@functools.cache
def _build(batch: int, seq_len: int, d: int):
    assert seq_len % NUM_WORKERS == 0
    span = seq_len // NUM_WORKERS
    assert span % CHUNK == 0 and d % LANES == 0
    n_sub = span // CHUNK
    n_chunks = batch * n_sub
    assert n_chunks >= NBUF

    mesh = plsc.VectorSubcoreMesh(core_axis_name="c", subcore_axis_name="s")

    scratch = (
        [pltpu.VMEM((CHUNK,), jnp.int32) for _ in range(NBUF)]
        + [pltpu.VMEM((CHUNK, d), jnp.float32) for _ in range(NBUF)]
        + [pltpu.VMEM((span + 8, d), jnp.float32)]
        + [pltpu.VMEM((CHUNK * LANES,), jnp.float32)]
        + [pltpu.SemaphoreType.DMA] * (2 * NBUF + 1)
    )

    @functools.partial(
        pl.kernel,
        out_type=jax.ShapeDtypeStruct((batch * seq_len, d), jnp.float32),
        mesh=mesh,
        scratch_types=scratch,
    )
    def k(enc_hbm, emb_hbm, pe_hbm, out_hbm, *bufs):
        ids_v = bufs[0:NBUF]
        tok_v = bufs[NBUF : 2 * NBUF]
        peb = bufs[2 * NBUF]
        mrep = bufs[2 * NBUF + 1]
        sems = bufs[2 * NBUF + 2 :]
        tok_sem = sems[0:NBUF]
        st_sem = sems[NBUF : 2 * NBUF]
        pe_sem = sems[2 * NBUF]

        wid = lax.axis_index("s") * NUM_CORES + lax.axis_index("c")
        l_base = wid * span

        pe_cp = pltpu.async_copy(pe_hbm.at[pl.ds(l_base, span + 8)], peb, pe_sem)

        def chunk_off(ci):
            b_row, s = ci // n_sub, ci % n_sub
            return b_row * seq_len + l_base + s * CHUNK

        tok_cp = [None] * n_chunks
        st_cp = [None] * n_chunks

        def issue_gather(ci):
            b = ci % NBUF
            pltpu.sync_copy(enc_hbm.at[pl.ds(chunk_off(ci), CHUNK)], ids_v[b])
            tok_cp[ci] = pltpu.async_copy(
                emb_hbm.at[ids_v[b]], tok_v[b], tok_sem[b]
            )

        issue_gather(0)
        pe_cp.wait()

        for ci in range(n_chunks):
            b = ci % NBUF
            s = ci % n_sub
            if ci + 1 < n_chunks:
                if st_cp[ci + 1 - NBUF] is not None:
                    st_cp[ci + 1 - NBUF].wait()
                issue_gather(ci + 1)
            tok_cp[ci].wait()

            def row_add(i, carry, _b=b, _s=s):
                for j in range(d // LANES):
                    sl = pl.ds(j * LANES, LANES)
                    tok_v[_b][i, sl] = (
                        tok_v[_b][i, sl] + peb[_s * CHUNK + i + 1, sl]
                    )
                return carry

            lax.fori_loop(0, CHUNK, row_add, 0)

            st_cp[ci] = pltpu.async_copy(
                tok_v[b], out_hbm.at[pl.ds(chunk_off(ci), CHUNK)], st_sem[b]
            )

        for ci in range(n_chunks - NBUF, n_chunks):
            st_cp[ci].wait()

    return k


def kernel(enc_input, src_emb, pe_table):
    b, l = enc_input.shape
    d = src_emb.shape[1]
    flat = enc_input.reshape(b * l)
    out = _build(b, l, d)(flat, src_emb, pe_table)
    return out.reshape(b, l, d)


# speedup vs baseline: 2.7085x; 1.7900x over previous
import functools

import jax
import jax.numpy as jnp
from jax import lax
from jax.experimental import pallas as pl
from jax.experimental.pallas import tpu as pltpu
from jax.experimental.pallas import tpu_sc as plsc

PADDING_ID = 0
NUM_CORES = 2
NUM_SUBCORES = 16
LANES = 16
NUM_WORKERS = NUM_CORES * NUM_SUBCORES
NBUF = 3
CHUNK = 16


@functools.cache
def _build(n_flat: int, seq_len: int, d: int):
    per_worker = n_flat // NUM_WORKERS
    assert n_flat % (NUM_WORKERS * CHUNK) == 0 and d % LANES == 0
    n_chunks = per_worker // CHUNK
    assert n_chunks >= NBUF
    assert seq_len % per_worker == 0 or per_worker % seq_len == 0

    mesh = plsc.VectorSubcoreMesh(core_axis_name="c", subcore_axis_name="s")

    scratch = (
        [pltpu.VMEM((per_worker,), jnp.int32)]
        + [pltpu.VMEM((per_worker,), jnp.int32)]
        + [pltpu.VMEM((CHUNK, d), jnp.float32) for _ in range(NBUF)]
        + [pltpu.VMEM((CHUNK, d), jnp.float32) for _ in range(NBUF)]
        + [pltpu.SemaphoreType.DMA] * (3 * NBUF)
    )

    @functools.partial(
        pl.kernel,
        out_type=jax.ShapeDtypeStruct((n_flat, d), jnp.float32),
        mesh=mesh,
        scratch_types=scratch,
    )
    def k(enc_hbm, emb_hbm, pe_hbm, out_hbm, *bufs):
        ids_all = bufs[0]
        pos_all = bufs[1]
        tok_v = bufs[2 : 2 + NBUF]
        pe_v = bufs[2 + NBUF : 2 + 2 * NBUF]
        sems = bufs[2 + 2 * NBUF :]
        tok_sem = sems[0:NBUF]
        pe_sem = sems[NBUF : 2 * NBUF]
        st_sem = sems[2 * NBUF : 3 * NBUF]

        wid = lax.axis_index("s") * NUM_CORES + lax.axis_index("c")
        base = wid * per_worker
        l0 = lax.rem(base, seq_len)

        pltpu.sync_copy(enc_hbm.at[pl.ds(base, per_worker)], ids_all)
        for j in range(per_worker // LANES):
            sl = pl.ds(j * LANES, LANES)
            ids = ids_all[sl]
            p = lax.broadcasted_iota(jnp.int32, (LANES,), 0) + (
                l0 + (j * LANES + 1)
            )
            pos_all[sl] = jnp.where(ids == PADDING_ID, 0, p)

        tok_cp = [None] * n_chunks
        pe_cp = [None] * n_chunks
        st_cp = [None] * n_chunks

        def issue_gather(ci):
            b = ci % NBUF
            sl = pl.ds(ci * CHUNK, CHUNK)
            tok_cp[ci] = pltpu.async_copy(
                emb_hbm.at[ids_all.at[sl]], tok_v[b], tok_sem[b]
            )
            pe_cp[ci] = pltpu.async_copy(
                pe_hbm.at[pos_all.at[sl]], pe_v[b], pe_sem[b]
            )

        for ci in range(NBUF - 1):
            issue_gather(ci)

        for ci in range(n_chunks):
            b = ci % NBUF
            if ci + NBUF - 1 < n_chunks:
                if st_cp[ci - 1] is not None:
                    st_cp[ci - 1].wait()
                issue_gather(ci + NBUF - 1)
            tok_cp[ci].wait()
            pe_cp[ci].wait()

            def row_add(i, carry, _b=b):
                for j in range(d // LANES):
                    sl = pl.ds(j * LANES, LANES)
                    tok_v[_b][i, sl] = tok_v[_b][i, sl] + pe_v[_b][i, sl]
                return carry

            lax.fori_loop(0, CHUNK, row_add, 0)
            off = base + ci * CHUNK
            st_cp[ci] = pltpu.async_copy(
                tok_v[b], out_hbm.at[pl.ds(off, CHUNK)], st_sem[b]
            )

        for ci in range(max(0, n_chunks - NBUF), n_chunks):
            if st_cp[ci] is not None:
                st_cp[ci].wait()

    return k


def kernel(enc_input, src_emb, pe_table):
    b, l = enc_input.shape
    d = src_emb.shape[1]
    flat = enc_input.reshape(b * l)
    out = _build(b * l, l, d)(flat, src_emb, pe_table)
    return out.reshape(b, l, d)


# speedup vs baseline: 2.7590x; 1.0187x over previous
import functools

import jax
import jax.numpy as jnp
from jax import lax
from jax.experimental import pallas as pl
from jax.experimental.pallas import tpu as pltpu
from jax.experimental.pallas import tpu_sc as plsc

PADDING_ID = 0
NUM_CORES = 2
NUM_SUBCORES = 16
LANES = 16
NUM_WORKERS = NUM_CORES * NUM_SUBCORES
NBUF = 3
CHUNK = 16


@functools.cache
def _build(n_flat: int, seq_len: int, d: int):
    per_worker = n_flat // NUM_WORKERS
    assert n_flat % (NUM_WORKERS * CHUNK) == 0 and d % LANES == 0
    n_chunks = per_worker // CHUNK
    assert n_chunks >= NBUF
    assert seq_len % per_worker == 0 or per_worker % seq_len == 0

    mesh = plsc.VectorSubcoreMesh(core_axis_name="c", subcore_axis_name="s")

    scratch = (
        [pltpu.VMEM((per_worker,), jnp.int32)]
        + [pltpu.VMEM((per_worker,), jnp.int32)]
        + [pltpu.VMEM((CHUNK, d), jnp.float32) for _ in range(NBUF)]
        + [pltpu.VMEM((CHUNK, d), jnp.float32) for _ in range(NBUF)]
        + [pltpu.SemaphoreType.DMA] * (3 * NBUF)
    )

    @functools.partial(
        pl.kernel,
        out_type=jax.ShapeDtypeStruct((n_flat, d), jnp.float32),
        mesh=mesh,
        scratch_types=scratch,
    )
    def k(enc_hbm, emb_hbm, pe_hbm, out_hbm, *bufs):
        ids_all = bufs[0]
        pos_all = bufs[1]
        tok_v = bufs[2 : 2 + NBUF]
        pe_v = bufs[2 + NBUF : 2 + 2 * NBUF]
        sems = bufs[2 + 2 * NBUF :]
        tok_sem = sems[0:NBUF]
        pe_sem = sems[NBUF : 2 * NBUF]
        st_sem = sems[2 * NBUF : 3 * NBUF]

        wid = lax.axis_index("s") * NUM_CORES + lax.axis_index("c")
        base = wid * per_worker
        l0 = lax.rem(base, seq_len)

        pltpu.sync_copy(enc_hbm.at[pl.ds(base, per_worker)], ids_all)
        for j in range(per_worker // LANES):
            sl = pl.ds(j * LANES, LANES)
            ids = ids_all[sl]
            p = lax.broadcasted_iota(jnp.int32, (LANES,), 0) + (
                l0 + (j * LANES + 1)
            )
            pos_all[sl] = jnp.where(ids == PADDING_ID, 0, p)

        tok_cp = [None] * n_chunks
        pe_cp = [None] * n_chunks
        st_cp = [None] * n_chunks

        def issue_gather(ci):
            b = ci % NBUF
            sl = pl.ds(ci * CHUNK, CHUNK)
            tok_cp[ci] = pltpu.async_copy(
                emb_hbm.at[ids_all.at[sl]], tok_v[b], tok_sem[b]
            )
            pe_cp[ci] = pltpu.async_copy(
                pe_hbm.at[pos_all.at[sl]], pe_v[b], pe_sem[b]
            )

        for ci in range(NBUF - 1):
            issue_gather(ci)

        for ci in range(n_chunks):
            b = ci % NBUF
            tok_cp[ci].wait()
            pe_cp[ci].wait()

            def row_add(i, carry, _b=b):
                for j in range(d // LANES):
                    sl = pl.ds(j * LANES, LANES)
                    tok_v[_b][i, sl] = tok_v[_b][i, sl] + pe_v[_b][i, sl]
                return carry

            lax.fori_loop(0, CHUNK, row_add, 0)
            if ci + NBUF - 1 < n_chunks:
                if st_cp[ci - 1] is not None:
                    st_cp[ci - 1].wait()
                issue_gather(ci + NBUF - 1)
            off = base + ci * CHUNK
            st_cp[ci] = pltpu.async_copy(
                tok_v[b], out_hbm.at[pl.ds(off, CHUNK)], st_sem[b]
            )

        for ci in range(max(0, n_chunks - NBUF), n_chunks):
            if st_cp[ci] is not None:
                st_cp[ci].wait()

    return k


def kernel(enc_input, src_emb, pe_table):
    b, l = enc_input.shape
    d = src_emb.shape[1]
    flat = enc_input.reshape(b * l)
    out = _build(b * l, l, d)(flat, src_emb, pe_table)
    return out.reshape(b, l, d)


# speedup vs baseline: 2.8372x; 1.0283x over previous
import functools

import jax
import jax.numpy as jnp
from jax import lax
from jax.experimental import pallas as pl
from jax.experimental.pallas import tpu as pltpu
from jax.experimental.pallas import tpu_sc as plsc

PADDING_ID = 0
NUM_CORES = 2
NUM_SUBCORES = 16
LANES = 16
NUM_WORKERS = NUM_CORES * NUM_SUBCORES
TBUF = 4
PBUF = 3
CHUNK = 16


@functools.cache
def _build(n_flat: int, seq_len: int, d: int):
    per_worker = n_flat // NUM_WORKERS
    assert n_flat % (NUM_WORKERS * CHUNK) == 0 and d % LANES == 0
    n_chunks = per_worker // CHUNK
    assert n_chunks >= TBUF
    assert seq_len % per_worker == 0

    mesh = plsc.VectorSubcoreMesh(core_axis_name="c", subcore_axis_name="s")

    scratch = (
        [pltpu.VMEM((per_worker,), jnp.int32)]
        + [pltpu.VMEM((per_worker,), jnp.int32)]
        + [pltpu.VMEM((CHUNK, d), jnp.float32) for _ in range(TBUF)]
        + [pltpu.VMEM((CHUNK, d), jnp.float32) for _ in range(PBUF)]
        + [pltpu.SemaphoreType.DMA] * (2 * TBUF + PBUF)
    )

    @functools.partial(
        pl.kernel,
        out_type=jax.ShapeDtypeStruct((n_flat, d), jnp.float32),
        mesh=mesh,
        scratch_types=scratch,
    )
    def k(enc_hbm, emb_hbm, pe_hbm, out_hbm, *bufs):
        ids_all = bufs[0]
        pos_all = bufs[1]
        tok_v = bufs[2 : 2 + TBUF]
        pe_v = bufs[2 + TBUF : 2 + TBUF + PBUF]
        sems = bufs[2 + TBUF + PBUF :]
        tok_sem = sems[0:TBUF]
        st_sem = sems[TBUF : 2 * TBUF]
        pe_sem = sems[2 * TBUF : 2 * TBUF + PBUF]

        wid = lax.axis_index("s") * NUM_CORES + lax.axis_index("c")
        base = wid * per_worker
        l0 = lax.rem(base, seq_len)

        pltpu.sync_copy(enc_hbm.at[pl.ds(base, per_worker)], ids_all)
        for j in range(per_worker // LANES):
            sl = pl.ds(j * LANES, LANES)
            ids = ids_all[sl]
            p = lax.broadcasted_iota(jnp.int32, (LANES,), 0) + (
                l0 + (j * LANES + 1)
            )
            pos_all[sl] = jnp.where(ids == PADDING_ID, 0, p)

        tok_cp = [None] * n_chunks
        pe_cp = [None] * n_chunks
        st_cp = [None] * n_chunks

        def issue_tok(ci):
            b = ci % TBUF
            tok_cp[ci] = pltpu.async_copy(
                emb_hbm.at[ids_all.at[pl.ds(ci * CHUNK, CHUNK)]],
                tok_v[b],
                tok_sem[b],
            )

        def issue_pe(ci):
            b = ci % PBUF
            pe_cp[ci] = pltpu.async_copy(
                pe_hbm.at[pos_all.at[pl.ds(ci * CHUNK, CHUNK)]],
                pe_v[b],
                pe_sem[b],
            )

        for ci in range(2):
            issue_tok(ci)
            issue_pe(ci)
        issue_tok(2)

        for ci in range(n_chunks):
            b = ci % TBUF
            tok_cp[ci].wait()
            pe_cp[ci].wait()
            if ci + 2 < n_chunks:
                issue_pe(ci + 2)

            def row_add(i, carry, _b=b, _p=ci % PBUF):
                for j in range(d // LANES):
                    sl = pl.ds(j * LANES, LANES)
                    tok_v[_b][i, sl] = tok_v[_b][i, sl] + pe_v[_p][i, sl]
                return carry

            lax.fori_loop(0, CHUNK, row_add, 0)
            if ci + 3 < n_chunks:
                if st_cp[ci - 1] is not None:
                    st_cp[ci - 1].wait()
                issue_tok(ci + 3)
            off = base + ci * CHUNK
            st_cp[ci] = pltpu.async_copy(
                tok_v[b], out_hbm.at[pl.ds(off, CHUNK)], st_sem[b]
            )

        for ci in range(max(0, n_chunks - TBUF), n_chunks):
            if st_cp[ci] is not None:
                st_cp[ci].wait()

    return k


def kernel(enc_input, src_emb, pe_table):
    b, l = enc_input.shape
    d = src_emb.shape[1]
    flat = enc_input.reshape(b * l)
    out = _build(b * l, l, d)(flat, src_emb, pe_table)
    return out.reshape(b, l, d)


# speedup vs baseline: 2.8401x; 1.0010x over previous
import functools

import jax
import jax.numpy as jnp
from jax import lax
from jax.experimental import pallas as pl
from jax.experimental.pallas import tpu as pltpu
from jax.experimental.pallas import tpu_sc as plsc

PADDING_ID = 0
NUM_CORES = 2
NUM_SUBCORES = 16
LANES = 16
NUM_WORKERS = NUM_CORES * NUM_SUBCORES
TBUF = 4
PBUF = 3
CHUNK = 16


@functools.cache
def _build(n_flat: int, seq_len: int, d: int):
    per_worker = n_flat // NUM_WORKERS
    assert n_flat % (NUM_WORKERS * CHUNK) == 0 and d % LANES == 0
    n_chunks = per_worker // CHUNK
    assert n_chunks >= TBUF
    assert seq_len % per_worker == 0

    mesh = plsc.VectorSubcoreMesh(core_axis_name="c", subcore_axis_name="s")

    scratch = (
        [pltpu.VMEM((per_worker,), jnp.int32)]
        + [pltpu.VMEM((per_worker,), jnp.int32)]
        + [pltpu.VMEM((CHUNK, d), jnp.float32) for _ in range(TBUF)]
        + [pltpu.VMEM((CHUNK, d), jnp.float32) for _ in range(PBUF)]
        + [pltpu.SemaphoreType.DMA] * (2 * TBUF + PBUF)
    )

    @functools.partial(
        pl.kernel,
        out_type=jax.ShapeDtypeStruct((n_flat, d), jnp.float32),
        mesh=mesh,
        scratch_types=scratch,
    )
    def k(enc_hbm, emb_hbm, pe_hbm, out_hbm, *bufs):
        ids_all = bufs[0]
        pos_all = bufs[1]
        tok_v = bufs[2 : 2 + TBUF]
        pe_v = bufs[2 + TBUF : 2 + TBUF + PBUF]
        sems = bufs[2 + TBUF + PBUF :]
        tok_sem = sems[0:TBUF]
        st_sem = sems[TBUF : 2 * TBUF]
        pe_sem = sems[2 * TBUF : 2 * TBUF + PBUF]

        wid = lax.axis_index("s") * NUM_CORES + lax.axis_index("c")
        base = wid * per_worker
        l0 = lax.rem(base, seq_len)

        pltpu.sync_copy(enc_hbm.at[pl.ds(base, per_worker)], ids_all)
        for j in range(per_worker // LANES):
            sl = pl.ds(j * LANES, LANES)
            ids = ids_all[sl]
            p = lax.broadcasted_iota(jnp.int32, (LANES,), 0) + (
                l0 + (j * LANES + 1)
            )
            pos_all[sl] = jnp.where(ids == PADDING_ID, 0, p)

        tok_cp = [None] * n_chunks
        pe_cp = [None] * n_chunks
        st_cp = [None] * n_chunks

        def issue_tok(ci):
            b = ci % TBUF
            tok_cp[ci] = pltpu.async_copy(
                emb_hbm.at[ids_all.at[pl.ds(ci * CHUNK, CHUNK)]],
                tok_v[b],
                tok_sem[b],
            )

        def issue_pe(ci):
            b = ci % PBUF
            pe_cp[ci] = pltpu.async_copy(
                pe_hbm.at[pos_all.at[pl.ds(ci * CHUNK, CHUNK)]],
                pe_v[b],
                pe_sem[b],
            )

        for ci in range(2):
            issue_tok(ci)
            issue_pe(ci)
        issue_tok(2)

        for ci in range(n_chunks):
            b = ci % TBUF
            if ci + 2 < n_chunks:
                issue_pe(ci + 2)
            tok_cp[ci].wait()
            pe_cp[ci].wait()

            def row_add(i, carry, _b=b, _p=ci % PBUF):
                for j in range(d // LANES):
                    sl = pl.ds(j * LANES, LANES)
                    tok_v[_b][i, sl] = tok_v[_b][i, sl] + pe_v[_p][i, sl]
                return carry

            lax.fori_loop(0, CHUNK, row_add, 0)
            if ci + 3 < n_chunks:
                if st_cp[ci - 1] is not None:
                    st_cp[ci - 1].wait()
                issue_tok(ci + 3)
            off = base + ci * CHUNK
            st_cp[ci] = pltpu.async_copy(
                tok_v[b], out_hbm.at[pl.ds(off, CHUNK)], st_sem[b]
            )

        for ci in range(max(0, n_chunks - TBUF), n_chunks):
            if st_cp[ci] is not None:
                st_cp[ci].wait()

    return k


def kernel(enc_input, src_emb, pe_table):
    b, l = enc_input.shape
    d = src_emb.shape[1]
    flat = enc_input.reshape(b * l)
    out = _build(b * l, l, d)(flat, src_emb, pe_table)
    return out.reshape(b, l, d)
